# baseline (speedup 1.0000x reference)
import functools

import jax
import jax.numpy as jnp
from jax import lax
from jax.experimental import pallas as pl
from jax.experimental.pallas import tpu as pltpu
from jax.experimental.pallas import tpu_sc as plsc

BATCH = 512
N = 50000
E = 800000
CHUNK = 32
ROWS = 2000
SEG_ROWS = 200
NEG_INF = float("-inf")

NT = N + 8
IW = 128
GH = 3
SB = 6
GPB = SB // GH
NGRP = 132
RPT = NGRP * GH
NSB = RPT // SB
EP = 16 * RPT * IW
_MESH = plsc.VectorSubcoreMesh(core_axis_name="c", subcore_axis_name="s")


def _relu(v):
    return jnp.maximum(v, 0.0)


def _mm_body(nchunk_in, nchunk_out, has_bias, *refs):
    n_in = nchunk_in if nchunk_in else 1
    ins = refs[:n_in]
    deg = refs[n_in]
    W = refs[n_in + 1]
    b = refs[n_in + 2] if has_bias else None
    outs = refs[n_in + 2 + (1 if has_bias else 0):]
    dinv = lax.rsqrt(deg[...][:, :1] + 1.0)
    Wv = W[...]
    if nchunk_in == 0:
        h = ins[0][...]
        th = jnp.dot(h, Wv, preferred_element_type=jnp.float32)
    else:
        bv = b[...]
        acc = None
        for c in range(n_in):
            hc = _relu(ins[c][...] * dinv + bv[:, c * CHUNK:(c + 1) * CHUNK])
            p = jnp.dot(hc, Wv[c * CHUNK:(c + 1) * CHUNK, :],
                        preferred_element_type=jnp.float32)
            acc = p if acc is None else acc + p
        th = acc
    th = th * dinv
    for c in range(nchunk_out):
        outs[c][...] = th[:, c * CHUNK:(c + 1) * CHUNK]


def _mm(x_or_chunks, deg, W, b, nchunk_in, nchunk_out):
    grid = (N // ROWS,)
    rspec = lambda w: pl.BlockSpec((ROWS, w), lambda i: (i, 0))
    wspec = pl.BlockSpec(W.shape, lambda i: (0, 0))
    if nchunk_in == 0:
        in_specs = [rspec(x_or_chunks.shape[1])]
        args = [x_or_chunks]
    else:
        in_specs = [rspec(CHUNK)] * nchunk_in
        args = list(x_or_chunks)
    in_specs += [rspec(16), wspec]
    args += [deg, W]
    has_bias = b is not None
    if has_bias:
        in_specs.append(pl.BlockSpec(b.shape, lambda i: (0, 0)))
        args.append(b)
    return pl.pallas_call(
        functools.partial(_mm_body, nchunk_in, nchunk_out, has_bias),
        grid=grid,
        in_specs=in_specs,
        out_specs=[rspec(CHUNK)] * nchunk_out,
        out_shape=[jax.ShapeDtypeStruct((N, CHUNK), jnp.float32)] * nchunk_out,
    )(*args)


def _segmax_body(nchunk, *refs):
    aggs = refs[:nchunk]
    deg, batch, b3, out = refs[nchunk:nchunk + 4]

    @pl.when(pl.program_id(0) == 0)
    def _init():
        out[...] = jnp.full((BATCH, nchunk * CHUNK), NEG_INF, jnp.float32)

    dinv = lax.rsqrt(deg[...][:, :1] + 1.0)
    agg = jnp.concatenate([a[...] for a in aggs], axis=1)
    h3 = _relu(agg * dinv + b3[...])
    bcol = batch[...]
    s_lo = bcol[0, 0]
    s_hi = bcol[SEG_ROWS - 1, 0]
    g_lo = s_lo // 8

    def grp(j, _):
        g8 = (g_lo + j) * 8
        segs = g8 + lax.broadcasted_iota(jnp.int32, (1, 8), 1)
        mask = bcol == segs
        rows = []
        for s in range(8):
            vals = jnp.where(mask[:, s:s + 1], h3, NEG_INF)
            rows.append(jnp.max(vals, axis=0, keepdims=True))
        m8 = jnp.concatenate(rows, axis=0)
        cur = out[pl.ds(g8, 8), :]
        out[pl.ds(g8, 8), :] = jnp.maximum(cur, m8)
        return 0

    lax.fori_loop(0, s_hi // 8 - g_lo + 1, grp, 0)


def _segmax(agg_chunks, deg, batch_col, b3, nchunk):
    rspec = lambda w: pl.BlockSpec((SEG_ROWS, w), lambda i: (i, 0))
    width = nchunk * CHUNK
    return pl.pallas_call(
        functools.partial(_segmax_body, nchunk),
        grid=(N // SEG_ROWS,),
        in_specs=[rspec(CHUNK)] * nchunk + [
            rspec(16), rspec(1), pl.BlockSpec((1, width), lambda i: (0, 0))],
        out_specs=pl.BlockSpec((BATCH, width), lambda i: (0, 0)),
        out_shape=jax.ShapeDtypeStruct((BATCH, width), jnp.float32),
    )(*agg_chunks, deg, batch_col, b3)


def _head_body(sm1, sm2, cell,
               d1g1W, d1g1b, d1g2W, d1g2b,
               d2g1W, d2g1b, d2g2W, d2g2b,
               r1W, r1b, r2W, r2b, r3W, r3b,
               gW, gb, f1W, f1b, f2W, f2b, oW, ob,
               out):
    dot = lambda a, w: jnp.dot(a, w, preferred_element_type=jnp.float32)
    g1 = _relu(dot(sm1[...], d1g1W[...]) + d1g1b[...])
    g1 = dot(g1, d1g2W[...]) + d1g2b[...]
    g2 = _relu(dot(sm2[...], d2g1W[...]) + d2g1b[...])
    g2 = dot(g2, d2g2W[...]) + d2g2b[...]
    c = cell[...]
    nrm = jnp.sqrt(jnp.sum(c * c, axis=1, keepdims=True))
    cn = c / jnp.maximum(nrm, 1e-12)
    cv = _relu(dot(cn, r1W[...]) + r1b[...])
    cv = _relu(dot(cv, r2W[...]) + r2b[...])
    cv = dot(cv, r3W[...]) + r3b[...]
    inter = g1 * g2
    gate = jax.nn.sigmoid(dot(g1, gW[...][:128]) + dot(g2, gW[...][128:]) + gb[...])
    syn = gate * inter
    xc = jnp.concatenate([g1, g2, syn, cv], axis=1)
    xc = _relu(dot(xc, f1W[...]) + f1b[...])
    xc = _relu(dot(xc, f2W[...]) + f2b[...])
    out[...] = dot(xc, oW[...]) + ob[...]


def _head(sm1, sm2, cell, hw):
    return pl.pallas_call(
        _head_body,
        out_shape=jax.ShapeDtypeStruct((BATCH, 2), jnp.float32),
    )(sm1, sm2, cell, *hw)


def _rowsplit(copy):
    sid = lax.axis_index("s")

    @pl.when(sid < 15)
    def _main():
        copy(sid * 3128, 3128)

    @pl.when(sid == 15)
    def _tail():
        copy(46920, 3080)


def _drain(th_c, rows, sem, cnt):
    for j in range(cnt):
        pltpu.make_async_copy(th_c.at[pl.ds(0, IW)], rows.at[0, j], sem).wait()


def _sc_pass(th_c, out_c, src2d, dst2d, table, sidx, didx, rows, gsem, ssem):
    sid = lax.axis_index("s")
    _rowsplit(lambda r0, rn: pltpu.sync_copy(
        th_c.at[pl.ds(r0, rn)], table.at[pl.ds(r0, rn)]))
    plsc.subcore_barrier()

    def load_sb(sb):
        row0 = sid * RPT + sb * SB
        slot = sb % 2
        pltpu.sync_copy(src2d.at[pl.ds(row0, SB)], sidx.at[slot])
        pltpu.sync_copy(dst2d.at[pl.ds(row0, SB)], didx.at[slot])

    def fire_gather(g):
        sbp = (g // GPB) % 2
        r = (g % GPB) * GH
        h = g % 2
        for j in range(GH):
            pltpu.async_copy(th_c.at[sidx.at[sbp, r + j]], rows.at[h, j], gsem)

    def fire_scatter(g):
        sbp = (g // GPB) % 2
        r = (g % GPB) * GH
        h = g % 2
        for j in range(GH):
            pltpu.async_copy(rows.at[h, j], table.at[didx.at[sbp, r + j]],
                             ssem, add=True)

    load_sb(0)
    fire_gather(0)

    def body(g, carry):
        @pl.when(g > 0)
        def _drain_prev_scatter():
            _drain(th_c, rows, ssem, GH)

        @pl.when(g + 1 < NGRP)
        def _next_gather():
            @pl.when((g + 1) % GPB == 0)
            def _load():
                load_sb((g + 1) // GPB)

            fire_gather(g + 1)

        _drain(th_c, rows, gsem, GH)
        fire_scatter(g)
        return carry

    lax.fori_loop(0, NGRP, body, 0)
    _drain(th_c, rows, ssem, GH)
    plsc.subcore_barrier()
    _rowsplit(lambda r0, rn: pltpu.sync_copy(
        table.at[pl.ds(r0, rn)], out_c.at[pl.ds(r0, rn)]))
    plsc.subcore_barrier()


def _agg_sc_body(nchunk, *refs):
    th1 = refs[:nchunk]
    th2 = refs[nchunk:2 * nchunk]
    src1, dst1, src2, dst2 = refs[2 * nchunk:2 * nchunk + 4]
    outs1 = refs[2 * nchunk + 4:3 * nchunk + 4]
    outs2 = refs[3 * nchunk + 4:4 * nchunk + 4]
    table, sidx, didx, rows, gsem, ssem = refs[4 * nchunk + 4:]
    cid = lax.axis_index("c")
    for cv in range(2):
        @pl.when(cid == cv)
        def _core(cv=cv):
            th = (th1, th2)[cv]
            outs = (outs1, outs2)[cv]
            src2d = (src1, src2)[cv]
            dst2d = (dst1, dst2)[cv]
            for c in range(nchunk):
                _sc_pass(th[c], outs[c], src2d, dst2d,
                         table, sidx, didx, rows, gsem, ssem)


def _aggregate_sc(th1_chunks, th2_chunks, e1, e2):
    n = len(th1_chunks)
    outs = pl.kernel(
        functools.partial(_agg_sc_body, n),
        out_type=[jax.ShapeDtypeStruct((N, CHUNK), jnp.float32)] * (2 * n),
        mesh=_MESH,
        compiler_params=pltpu.CompilerParams(use_tc_tiling_on_sc=False),
        scratch_types=[
            pltpu.VMEM_SHARED((NT, CHUNK), jnp.float32),
            pltpu.VMEM((2, SB, IW), jnp.int32),
            pltpu.VMEM((2, SB, IW), jnp.int32),
            pltpu.VMEM((2, GH, IW, CHUNK), jnp.float32),
            pltpu.SemaphoreType.DMA,
            pltpu.SemaphoreType.DMA,
        ],
    )(*th1_chunks, *th2_chunks, e1[0], e1[1], e2[0], e2[1])
    return list(outs[:n]), list(outs[n:])


def _deg_sc_body(dst2d_1, dst2d_2, zeros16, ones_in,
                 deg1, deg2, table, didx, ones_v, ssem):
    cid = lax.axis_index("c")
    sid = lax.axis_index("s")
    pltpu.sync_copy(ones_in, ones_v)
    for cv in range(2):
        @pl.when(cid == cv)
        def _core(cv=cv):
            dst2d = (dst2d_1, dst2d_2)[cv]
            out = (deg1, deg2)[cv]
            _rowsplit(lambda r0, rn: pltpu.sync_copy(
                zeros16.at[pl.ds(r0, rn)], table.at[pl.ds(r0, rn)]))
            plsc.subcore_barrier()

            def dsb(sb, carry):
                slot = sb % 2
                pltpu.sync_copy(dst2d.at[pl.ds(sid * RPT + sb * SB, SB)],
                                didx.at[slot])

                @pl.when(sb > 0)
                def _drain_prev():
                    for j in range(SB):
                        pltpu.make_async_copy(
                            zeros16.at[pl.ds(0, IW)], ones_v, ssem).wait()

                for j in range(SB):
                    pltpu.async_copy(ones_v, table.at[didx.at[slot, j]],
                                     ssem, add=True)
                return carry

            lax.fori_loop(0, NSB, dsb, 0)
            for j in range(SB):
                pltpu.make_async_copy(
                    zeros16.at[pl.ds(0, IW)], ones_v, ssem).wait()
            plsc.subcore_barrier()
            _rowsplit(lambda r0, rn: pltpu.sync_copy(
                table.at[pl.ds(r0, rn)], out.at[pl.ds(r0, rn)]))
            plsc.subcore_barrier()


def _deg_sc(e1, e2):
    zeros16 = jnp.zeros((N, 16), jnp.float32)
    ones_in = jnp.ones((IW, 16), jnp.float32)
    return pl.kernel(
        _deg_sc_body,
        out_type=[jax.ShapeDtypeStruct((N, 16), jnp.float32)] * 2,
        mesh=_MESH,
        compiler_params=pltpu.CompilerParams(use_tc_tiling_on_sc=False),
        scratch_types=[
            pltpu.VMEM_SHARED((NT, 16), jnp.float32),
            pltpu.VMEM((2, SB, IW), jnp.int32),
            pltpu.VMEM((IW, 16), jnp.float32),
            pltpu.SemaphoreType.DMA,
        ],
    )(e1[1], e2[1], zeros16, ones_in)


def _prep_edges(ei):
    src = jnp.concatenate([ei[0], jnp.zeros((EP - E,), jnp.int32)])
    dst = jnp.concatenate([ei[1], jnp.full((EP - E,), N, jnp.int32)])
    return src.reshape(EP // IW, IW), dst.reshape(EP // IW, IW)


def _pad_w(W, rows, cols):
    return jnp.pad(W, ((0, rows - W.shape[0]), (0, cols - W.shape[1])))


def _pad_b(b, cols):
    return jnp.pad(b, (0, cols - b.shape[0])).reshape(1, cols)


def _dual_branch(x1, x2, e1, e2, deg1, deg2, batch1, batch2, p1, p2):
    (W1a, b1a, W2a, b2a, W3a, b3a) = p1
    (W1b, b1b, W2b, b2b, W3b, b3b) = p2
    pads = lambda W1, b1, W2, b2, W3, b3: (
        _pad_w(W1, 78, 96), _pad_b(b1, 96), _pad_w(W2, 96, 160),
        _pad_b(b2, 160), _pad_w(W3, 160, 320), _pad_b(b3, 320))
    W1pa, b1pa, W2pa, b2pa, W3pa, b3pa = pads(W1a, b1a, W2a, b2a, W3a, b3a)
    W1pb, b1pb, W2pb, b2pb, W3pb, b3pb = pads(W1b, b1b, W2b, b2b, W3b, b3b)

    th0a = _mm(x1, deg1, W1pa, None, 0, 3)
    th0b = _mm(x2, deg2, W1pb, None, 0, 3)
    agg0a, agg0b = _aggregate_sc(th0a, th0b, e1, e2)
    th1a = _mm(agg0a, deg1, W2pa, b1pa, 3, 5)
    th1b = _mm(agg0b, deg2, W2pb, b1pb, 3, 5)
    agg1a, agg1b = _aggregate_sc(th1a, th1b, e1, e2)
    th2a = _mm(agg1a, deg1, W3pa, b2pa, 5, 10)
    th2b = _mm(agg1b, deg2, W3pb, b2pb, 5, 10)
    agg2a, agg2b = _aggregate_sc(th2a, th2b, e1, e2)
    sm1 = _segmax(agg2a, deg1, batch1.reshape(N, 1), b3pa, 10)
    sm2 = _segmax(agg2b, deg2, batch2.reshape(N, 1), b3pb, 10)
    return sm1[:, :312], sm2[:, :312]


def kernel(x1, edge_index1, batch1, cell, x2, edge_index2, batch2, params):
    (d1W1, d1b1, d1W2, d1b2, d1W3, d1b3, d1g1W, d1g1b, d1g2W, d1g2b,
     d2W1, d2b1, d2W2, d2b2, d2W3, d2b3, d2g1W, d2g1b, d2g2W, d2g2b,
     r1W, r1b, r2W, r2b, r3W, r3b, gW, gb, f1W, f1b, f2W, f2b, oW, ob) = params
    e1 = _prep_edges(edge_index1)
    e2 = _prep_edges(edge_index2)
    deg1, deg2 = _deg_sc(e1, e2)
    sm1, sm2 = _dual_branch(
        x1, x2, e1, e2, deg1, deg2, batch1, batch2,
        (d1W1, d1b1, d1W2, d1b2, d1W3, d1b3),
        (d2W1, d2b1, d2W2, d2b2, d2W3, d2b3))
    hw = (d1g1W, d1g1b, d1g2W, d1g2b, d2g1W, d2g1b, d2g2W, d2g2b,
          r1W, r1b, r2W, r2b, r3W, r3b, gW, gb, f1W, f1b, f2W, f2b, oW, ob)
    return _head(sm1, sm2, cell, hw)


# speedup vs baseline: 1.3407x; 1.3407x over previous
import functools

import jax
import jax.numpy as jnp
from jax import lax
from jax.experimental import pallas as pl
from jax.experimental.pallas import tpu as pltpu
from jax.experimental.pallas import tpu_sc as plsc

BATCH = 512
N = 50000
E = 800000
CHUNK = 32
ROWS = 2000
SEG_ROWS = 200
NEG_INF = float("-inf")

NT = N + 8
IW = 128
GH = 2
SB = 28
GPB = SB // GH
NGRP = 196
RPT = NGRP * GH
NSB = RPT // SB
EP = 16 * RPT * IW
_MESH = plsc.VectorSubcoreMesh(core_axis_name="c", subcore_axis_name="s")


def _relu(v):
    return jnp.maximum(v, 0.0)


def _mm_body(nchunk_in, nchunk_out, has_bias, *refs):
    n_in = nchunk_in if nchunk_in else 1
    ins = refs[:n_in]
    deg = refs[n_in]
    W = refs[n_in + 1]
    b = refs[n_in + 2] if has_bias else None
    outs = refs[n_in + 2 + (1 if has_bias else 0):]
    dinv = lax.rsqrt(deg[...][:, :1] + 1.0)
    Wv = W[...]
    if nchunk_in == 0:
        h = ins[0][...]
        th = jnp.dot(h, Wv, preferred_element_type=jnp.float32)
    else:
        bv = b[...]
        acc = None
        for c in range(n_in):
            hc = _relu(ins[c][...] * dinv + bv[:, c * CHUNK:(c + 1) * CHUNK])
            p = jnp.dot(hc, Wv[c * CHUNK:(c + 1) * CHUNK, :],
                        preferred_element_type=jnp.float32)
            acc = p if acc is None else acc + p
        th = acc
    th = th * dinv
    for c in range(nchunk_out):
        outs[c][...] = th[:, c * CHUNK:(c + 1) * CHUNK]


def _mm(x_or_chunks, deg, W, b, nchunk_in, nchunk_out):
    grid = (N // ROWS,)
    rspec = lambda w: pl.BlockSpec((ROWS, w), lambda i: (i, 0))
    wspec = pl.BlockSpec(W.shape, lambda i: (0, 0))
    if nchunk_in == 0:
        in_specs = [rspec(x_or_chunks.shape[1])]
        args = [x_or_chunks]
    else:
        in_specs = [rspec(CHUNK)] * nchunk_in
        args = list(x_or_chunks)
    in_specs += [rspec(16), wspec]
    args += [deg, W]
    has_bias = b is not None
    if has_bias:
        in_specs.append(pl.BlockSpec(b.shape, lambda i: (0, 0)))
        args.append(b)
    return pl.pallas_call(
        functools.partial(_mm_body, nchunk_in, nchunk_out, has_bias),
        grid=grid,
        in_specs=in_specs,
        out_specs=[rspec(CHUNK)] * nchunk_out,
        out_shape=[jax.ShapeDtypeStruct((N, CHUNK), jnp.float32)] * nchunk_out,
    )(*args)


def _segmax_body(nchunk, *refs):
    aggs = refs[:nchunk]
    deg, batch, b3, out = refs[nchunk:nchunk + 4]

    @pl.when(pl.program_id(0) == 0)
    def _init():
        out[...] = jnp.full((BATCH, nchunk * CHUNK), NEG_INF, jnp.float32)

    dinv = lax.rsqrt(deg[...][:, :1] + 1.0)
    agg = jnp.concatenate([a[...] for a in aggs], axis=1)
    h3 = _relu(agg * dinv + b3[...])
    bcol = batch[...]
    s_lo = bcol[0, 0]
    s_hi = bcol[SEG_ROWS - 1, 0]
    g_lo = s_lo // 8

    def grp(j, _):
        g8 = (g_lo + j) * 8
        segs = g8 + lax.broadcasted_iota(jnp.int32, (1, 8), 1)
        mask = bcol == segs
        rows = []
        for s in range(8):
            vals = jnp.where(mask[:, s:s + 1], h3, NEG_INF)
            rows.append(jnp.max(vals, axis=0, keepdims=True))
        m8 = jnp.concatenate(rows, axis=0)
        cur = out[pl.ds(g8, 8), :]
        out[pl.ds(g8, 8), :] = jnp.maximum(cur, m8)
        return 0

    lax.fori_loop(0, s_hi // 8 - g_lo + 1, grp, 0)


def _segmax(agg_chunks, deg, batch_col, b3, nchunk):
    rspec = lambda w: pl.BlockSpec((SEG_ROWS, w), lambda i: (i, 0))
    width = nchunk * CHUNK
    return pl.pallas_call(
        functools.partial(_segmax_body, nchunk),
        grid=(N // SEG_ROWS,),
        in_specs=[rspec(CHUNK)] * nchunk + [
            rspec(16), rspec(1), pl.BlockSpec((1, width), lambda i: (0, 0))],
        out_specs=pl.BlockSpec((BATCH, width), lambda i: (0, 0)),
        out_shape=jax.ShapeDtypeStruct((BATCH, width), jnp.float32),
    )(*agg_chunks, deg, batch_col, b3)


def _head_body(sm1, sm2, cell,
               d1g1W, d1g1b, d1g2W, d1g2b,
               d2g1W, d2g1b, d2g2W, d2g2b,
               r1W, r1b, r2W, r2b, r3W, r3b,
               gW, gb, f1W, f1b, f2W, f2b, oW, ob,
               out):
    dot = lambda a, w: jnp.dot(a, w, preferred_element_type=jnp.float32)
    g1 = _relu(dot(sm1[...], d1g1W[...]) + d1g1b[...])
    g1 = dot(g1, d1g2W[...]) + d1g2b[...]
    g2 = _relu(dot(sm2[...], d2g1W[...]) + d2g1b[...])
    g2 = dot(g2, d2g2W[...]) + d2g2b[...]
    c = cell[...]
    nrm = jnp.sqrt(jnp.sum(c * c, axis=1, keepdims=True))
    cn = c / jnp.maximum(nrm, 1e-12)
    cv = _relu(dot(cn, r1W[...]) + r1b[...])
    cv = _relu(dot(cv, r2W[...]) + r2b[...])
    cv = dot(cv, r3W[...]) + r3b[...]
    inter = g1 * g2
    gate = jax.nn.sigmoid(dot(g1, gW[...][:128]) + dot(g2, gW[...][128:]) + gb[...])
    syn = gate * inter
    xc = jnp.concatenate([g1, g2, syn, cv], axis=1)
    xc = _relu(dot(xc, f1W[...]) + f1b[...])
    xc = _relu(dot(xc, f2W[...]) + f2b[...])
    out[...] = dot(xc, oW[...]) + ob[...]


def _head(sm1, sm2, cell, hw):
    return pl.pallas_call(
        _head_body,
        out_shape=jax.ShapeDtypeStruct((BATCH, 2), jnp.float32),
    )(sm1, sm2, cell, *hw)


def _rowsplit(copy):
    sid = lax.axis_index("s")

    @pl.when(sid < 15)
    def _main():
        copy(sid * 3128, 3128)

    @pl.when(sid == 15)
    def _tail():
        copy(46920, 3080)


def _drain(th_c, rows, sem, cnt):
    for j in range(cnt):
        pltpu.make_async_copy(th_c.at[pl.ds(0, IW)], rows.at[0, j], sem).wait()


def _sc_pass(th_c, out_c, src2d, dst2d, table, sidx, didx, rows, gsem, ssem):
    sid = lax.axis_index("s")
    _rowsplit(lambda r0, rn: pltpu.sync_copy(
        th_c.at[pl.ds(r0, rn)], table.at[pl.ds(r0, rn)]))
    plsc.subcore_barrier()

    def load_sb(sb):
        row0 = sid * RPT + sb * SB
        slot = sb % 2
        pltpu.sync_copy(src2d.at[pl.ds(row0, SB)], sidx.at[slot])
        pltpu.sync_copy(dst2d.at[pl.ds(row0, SB)], didx.at[slot])

    def fire_gather(g):
        sbp = (g // GPB) % 2
        r = (g % GPB) * GH
        h = g % 2
        for j in range(GH):
            pltpu.async_copy(th_c.at[sidx.at[sbp, r + j]], rows.at[h, j], gsem)

    def fire_scatter(g):
        sbp = (g // GPB) % 2
        r = (g % GPB) * GH
        h = g % 2
        for j in range(GH):
            pltpu.async_copy(rows.at[h, j], table.at[didx.at[sbp, r + j]],
                             ssem, add=True)

    load_sb(0)
    fire_gather(0)

    def body(g, carry):
        @pl.when(g > 0)
        def _drain_prev_scatter():
            _drain(th_c, rows, ssem, GH)

        @pl.when(g + 1 < NGRP)
        def _next_gather():
            @pl.when((g + 1) % GPB == 0)
            def _load():
                load_sb((g + 1) // GPB)

            fire_gather(g + 1)

        _drain(th_c, rows, gsem, GH)
        fire_scatter(g)
        return carry

    lax.fori_loop(0, NGRP, body, 0)
    _drain(th_c, rows, ssem, GH)
    plsc.subcore_barrier()
    _rowsplit(lambda r0, rn: pltpu.sync_copy(
        table.at[pl.ds(r0, rn)], out_c.at[pl.ds(r0, rn)]))
    plsc.subcore_barrier()


def _agg_sc_body(nchunk, *refs):
    th1 = refs[:nchunk]
    th2 = refs[nchunk:2 * nchunk]
    src1, dst1, src2, dst2 = refs[2 * nchunk:2 * nchunk + 4]
    outs1 = refs[2 * nchunk + 4:3 * nchunk + 4]
    outs2 = refs[3 * nchunk + 4:4 * nchunk + 4]
    table, sidx, didx, rows, gsem, ssem = refs[4 * nchunk + 4:]
    cid = lax.axis_index("c")
    for cv in range(2):
        @pl.when(cid == cv)
        def _core(cv=cv):
            th = (th1, th2)[cv]
            outs = (outs1, outs2)[cv]
            src2d = (src1, src2)[cv]
            dst2d = (dst1, dst2)[cv]
            for c in range(nchunk):
                _sc_pass(th[c], outs[c], src2d, dst2d,
                         table, sidx, didx, rows, gsem, ssem)


def _aggregate_sc(th1_chunks, th2_chunks, e1, e2):
    n = len(th1_chunks)
    outs = pl.kernel(
        functools.partial(_agg_sc_body, n),
        out_type=[jax.ShapeDtypeStruct((N, CHUNK), jnp.float32)] * (2 * n),
        mesh=_MESH,
        compiler_params=pltpu.CompilerParams(use_tc_tiling_on_sc=False),
        scratch_types=[
            pltpu.VMEM_SHARED((NT, CHUNK), jnp.float32),
            pltpu.VMEM((2, SB, IW), jnp.int32),
            pltpu.VMEM((2, SB, IW), jnp.int32),
            pltpu.VMEM((2, GH, IW, CHUNK), jnp.float32),
            pltpu.SemaphoreType.DMA,
            pltpu.SemaphoreType.DMA,
        ],
    )(*th1_chunks, *th2_chunks, e1[0], e1[1], e2[0], e2[1])
    return list(outs[:n]), list(outs[n:])


def _deg_sc_body(dst2d_1, dst2d_2, zeros16, ones_in,
                 deg1, deg2, table, didx, ones_v, ssem):
    cid = lax.axis_index("c")
    sid = lax.axis_index("s")
    pltpu.sync_copy(ones_in, ones_v)
    for cv in range(2):
        @pl.when(cid == cv)
        def _core(cv=cv):
            dst2d = (dst2d_1, dst2d_2)[cv]
            out = (deg1, deg2)[cv]
            _rowsplit(lambda r0, rn: pltpu.sync_copy(
                zeros16.at[pl.ds(r0, rn)], table.at[pl.ds(r0, rn)]))
            plsc.subcore_barrier()

            def dsb(sb, carry):
                slot = sb % 2
                pltpu.sync_copy(dst2d.at[pl.ds(sid * RPT + sb * SB, SB)],
                                didx.at[slot])

                @pl.when(sb > 0)
                def _drain_prev():
                    for j in range(SB):
                        pltpu.make_async_copy(
                            zeros16.at[pl.ds(0, IW)], ones_v, ssem).wait()

                for j in range(SB):
                    pltpu.async_copy(ones_v, table.at[didx.at[slot, j]],
                                     ssem, add=True)
                return carry

            lax.fori_loop(0, NSB, dsb, 0)
            for j in range(SB):
                pltpu.make_async_copy(
                    zeros16.at[pl.ds(0, IW)], ones_v, ssem).wait()
            plsc.subcore_barrier()
            _rowsplit(lambda r0, rn: pltpu.sync_copy(
                table.at[pl.ds(r0, rn)], out.at[pl.ds(r0, rn)]))
            plsc.subcore_barrier()


def _deg_sc(e1, e2):
    zeros16 = jnp.zeros((N, 16), jnp.float32)
    ones_in = jnp.ones((IW, 16), jnp.float32)
    return pl.kernel(
        _deg_sc_body,
        out_type=[jax.ShapeDtypeStruct((N, 16), jnp.float32)] * 2,
        mesh=_MESH,
        compiler_params=pltpu.CompilerParams(use_tc_tiling_on_sc=False),
        scratch_types=[
            pltpu.VMEM_SHARED((NT, 16), jnp.float32),
            pltpu.VMEM((2, SB, IW), jnp.int32),
            pltpu.VMEM((IW, 16), jnp.float32),
            pltpu.SemaphoreType.DMA,
        ],
    )(e1[1], e2[1], zeros16, ones_in)


def _prep_edges(ei):
    src = jnp.concatenate([ei[0], jnp.zeros((EP - E,), jnp.int32)])
    dst = jnp.concatenate([ei[1], jnp.full((EP - E,), N, jnp.int32)])
    return src.reshape(EP // IW, IW), dst.reshape(EP // IW, IW)


def _pad_w(W, rows, cols):
    return jnp.pad(W, ((0, rows - W.shape[0]), (0, cols - W.shape[1])))


def _pad_b(b, cols):
    return jnp.pad(b, (0, cols - b.shape[0])).reshape(1, cols)


def _dual_branch(x1, x2, e1, e2, deg1, deg2, batch1, batch2, p1, p2):
    (W1a, b1a, W2a, b2a, W3a, b3a) = p1
    (W1b, b1b, W2b, b2b, W3b, b3b) = p2
    pads = lambda W1, b1, W2, b2, W3, b3: (
        _pad_w(W1, 78, 96), _pad_b(b1, 96), _pad_w(W2, 96, 160),
        _pad_b(b2, 160), _pad_w(W3, 160, 320), _pad_b(b3, 320))
    W1pa, b1pa, W2pa, b2pa, W3pa, b3pa = pads(W1a, b1a, W2a, b2a, W3a, b3a)
    W1pb, b1pb, W2pb, b2pb, W3pb, b3pb = pads(W1b, b1b, W2b, b2b, W3b, b3b)

    th0a = _mm(x1, deg1, W1pa, None, 0, 3)
    th0b = _mm(x2, deg2, W1pb, None, 0, 3)
    agg0a, agg0b = _aggregate_sc(th0a, th0b, e1, e2)
    th1a = _mm(agg0a, deg1, W2pa, b1pa, 3, 5)
    th1b = _mm(agg0b, deg2, W2pb, b1pb, 3, 5)
    agg1a, agg1b = _aggregate_sc(th1a, th1b, e1, e2)
    th2a = _mm(agg1a, deg1, W3pa, b2pa, 5, 10)
    th2b = _mm(agg1b, deg2, W3pb, b2pb, 5, 10)
    agg2a, agg2b = _aggregate_sc(th2a, th2b, e1, e2)
    sm1 = _segmax(agg2a, deg1, batch1.reshape(N, 1), b3pa, 10)
    sm2 = _segmax(agg2b, deg2, batch2.reshape(N, 1), b3pb, 10)
    return sm1[:, :312], sm2[:, :312]


def kernel(x1, edge_index1, batch1, cell, x2, edge_index2, batch2, params):
    (d1W1, d1b1, d1W2, d1b2, d1W3, d1b3, d1g1W, d1g1b, d1g2W, d1g2b,
     d2W1, d2b1, d2W2, d2b2, d2W3, d2b3, d2g1W, d2g1b, d2g2W, d2g2b,
     r1W, r1b, r2W, r2b, r3W, r3b, gW, gb, f1W, f1b, f2W, f2b, oW, ob) = params
    e1 = _prep_edges(edge_index1)
    e2 = _prep_edges(edge_index2)
    deg1, deg2 = _deg_sc(e1, e2)
    sm1, sm2 = _dual_branch(
        x1, x2, e1, e2, deg1, deg2, batch1, batch2,
        (d1W1, d1b1, d1W2, d1b2, d1W3, d1b3),
        (d2W1, d2b1, d2W2, d2b2, d2W3, d2b3))
    hw = (d1g1W, d1g1b, d1g2W, d1g2b, d2g1W, d2g1b, d2g2W, d2g2b,
          r1W, r1b, r2W, r2b, r3W, r3b, gW, gb, f1W, f1b, f2W, f2b, oW, ob)
    return _head(sm1, sm2, cell, hw)
